# Initial kernel scaffold; baseline (speedup 1.0000x reference)
import jax
import jax.numpy as jnp
from jax.experimental import pallas as pl


def kernel(boxes, scores, roi_features, pca_mean, pca_components, max_length, kde_memory, kde_bandwidth):
    raise NotImplementedError("write your pallas kernel here")


# speedup vs baseline: 15.8108x; 15.8108x over previous
import functools

import jax
import jax.numpy as jnp
import numpy as np
from jax import lax
from jax.experimental import pallas as pl
from jax.experimental.pallas import tpu as pltpu
from jax.experimental.pallas import tpu_sc as plsc

NBOX = 5000
NPAD = 5120
CHUNK = 320
NVR = CHUNK // 16
NSUB = 16
NKEEP = 100
KPAD = 128
FEAT = 4096
NTRAIN = 40000
NTPAD = 40960
KCH = 4096
NPCA = 16
NEG = -1e30


def _sc_nms_gather(scores_p, x1p, y1p, x2p, y2p, roi):
    mesh = plsc.VectorSubcoreMesh(
        core_axis_name="c", subcore_axis_name="s", num_cores=1, num_subcores=NSUB
    )

    @functools.partial(
        pl.kernel,
        out_type=[
            jax.ShapeDtypeStruct((KPAD, FEAT), jnp.float32),
            jax.ShapeDtypeStruct((KPAD,), jnp.float32),
            jax.ShapeDtypeStruct((KPAD * 4,), jnp.float32),
        ],
        mesh=mesh,
        compiler_params=pltpu.CompilerParams(needs_layout_passes=False),
        scratch_types=[
            pltpu.VMEM((NPAD,), jnp.float32),
            pltpu.VMEM((NPAD,), jnp.float32),
            pltpu.VMEM((NPAD,), jnp.float32),
            pltpu.VMEM((NPAD,), jnp.float32),
            pltpu.VMEM((CHUNK,), jnp.float32),
            pltpu.VMEM((CHUNK,), jnp.float32),
            pltpu.VMEM((CHUNK,), jnp.float32),
            pltpu.VMEM((CHUNK,), jnp.float32),
            pltpu.VMEM((CHUNK,), jnp.float32),
            pltpu.VMEM((KPAD,), jnp.int32),
            pltpu.VMEM((KPAD,), jnp.float32),
            pltpu.VMEM((KPAD * 4,), jnp.float32),
            pltpu.VMEM_SHARED((NSUB * 16,), jnp.float32),
            pltpu.VMEM_SHARED((NSUB * 16,), jnp.float32),
            pltpu.VMEM_SHARED((NSUB * 16,), jnp.float32),
            pltpu.VMEM_SHARED((NSUB * 16,), jnp.float32),
            pltpu.VMEM((16,), jnp.float32),
            pltpu.VMEM((16,), jnp.float32),
            pltpu.VMEM((NSUB * 16,), jnp.float32),
            pltpu.VMEM((NSUB * 16,), jnp.float32),
            pltpu.VMEM((8, FEAT), jnp.float32),
            pltpu.SemaphoreType.DMA,
        ],
    )
    def k(scores_h, x1_h, y1_h, x2_h, y2_h, roi_h,
          feats_o, kval_o, kbox_o,
          x1f, y1f, x2f, y2f, x1c, y1c, x2c, y2c, msc,
          kidx, kval, kbox, redv0, redi0, redv1, redi1, wv, wi, rv2, ri2, rows, sem):
        wid = lax.axis_index("s")
        base = pl.multiple_of(wid * CHUNK, CHUNK)

        pltpu.sync_copy(x1_h, x1f)
        pltpu.sync_copy(y1_h, y1f)
        pltpu.sync_copy(x2_h, x2f)
        pltpu.sync_copy(y2_h, y2f)
        pltpu.sync_copy(scores_h.at[pl.ds(base, CHUNK)], msc)
        pltpu.sync_copy(x1_h.at[pl.ds(base, CHUNK)], x1c)
        pltpu.sync_copy(y1_h.at[pl.ds(base, CHUNK)], y1c)
        pltpu.sync_copy(x2_h.at[pl.ds(base, CHUNK)], x2c)
        pltpu.sync_copy(y2_h.at[pl.ds(base, CHUNK)], y2c)

        iota_i = lax.iota(jnp.int32, 16)
        iota_f = iota_i.astype(jnp.float32)
        zeros_i = jnp.zeros((16,), jnp.int32)
        base_f = lax.convert_element_type(base, jnp.float32)

        for j in range(NVR):
            sl = pl.ds(16 * j, 16)
            s = msc[sl]
            w = x2c[sl] - x1c[sl]
            h = y2c[sl] - y1c[sl]
            ok = (s >= jnp.float32(0.001)) & (w >= jnp.float32(100.0)) & (h >= jnp.float32(100.0))
            msc[sl] = jnp.where(ok, s, NEG)

        zf = jnp.zeros((16,), jnp.float32)
        zi = jnp.zeros((16,), jnp.int32)
        for kk in range(KPAD // 16):
            kval[pl.ds(16 * kk, 16)] = zf
            kidx[pl.ds(16 * kk, 16)] = zi
        for kk in range(KPAD * 4 // 16):
            kbox[pl.ds(16 * kk, 16)] = zf

        def one_step(t, redv, redi):
            bv = jnp.full((16,), jnp.float32(-3e38))
            bj = zf
            for j in range(NVR):
                v = msc[pl.ds(16 * j, 16)]
                take = v > bv
                bv = jnp.where(take, v, bv)
                bj = jnp.where(take, jnp.full((16,), jnp.float32(j)), bj)
            lm = jnp.max(bv)
            gidx = base_f + bj * jnp.float32(16.0) + iota_f
            li = jnp.min(jnp.where(bv == lm, gidx, jnp.float32(3e38)))

            wv[...] = jnp.full((16,), lm)
            wi[...] = jnp.full((16,), li)
            woff = pl.multiple_of(wid * 16, 16)
            pltpu.sync_copy(wv, redv.at[pl.ds(woff, 16)])
            pltpu.sync_copy(wi, redi.at[pl.ds(woff, 16)])
            plsc.subcore_barrier()
            pltpu.sync_copy(redv, rv2)
            pltpu.sync_copy(redi, ri2)
            gv = plsc.load_gather(rv2, [iota_i * 16])
            gi = plsc.load_gather(ri2, [iota_i * 16])
            G = jnp.max(gv)
            If = jnp.min(jnp.where(gv == G, gi, jnp.float32(3e38)))
            ii = If.astype(jnp.int32)
            ii = lax.max(0, lax.min(ii, NPAD - 1))
            iiv = jnp.full((16,), ii)

            bx1 = plsc.load_gather(x1f, [iiv])
            by1 = plsc.load_gather(y1f, [iiv])
            bx2 = plsc.load_gather(x2f, [iiv])
            by2 = plsc.load_gather(y2f, [iiv])

            validf = jnp.where(G > jnp.float32(-5e29), jnp.float32(1.0), jnp.float32(0.0))
            tv = jnp.full((16,), t)
            plsc.store_scatter(kval, [tv], jnp.full((16,), validf), mask=iota_i == 0)
            plsc.store_scatter(kidx, [tv], iiv, mask=iota_i == 0)
            cvec = jnp.where(iota_i == 0, bx1,
                             jnp.where(iota_i == 1, by1,
                                       jnp.where(iota_i == 2, bx2, by2)))
            plsc.store_scatter(kbox, [tv * 4 + iota_i], cvec, mask=iota_i < 4)

            a1 = (bx2 - bx1) * (by2 - by1)
            for j in range(NVR):
                sl = pl.ds(16 * j, 16)
                xa = x1c[sl]
                ya = y1c[sl]
                xb = x2c[sl]
                yb = y2c[sl]
                m = msc[sl]
                xx1 = jnp.maximum(bx1, xa)
                yy1 = jnp.maximum(by1, ya)
                xx2 = jnp.minimum(bx2, xb)
                yy2 = jnp.minimum(by2, yb)
                inter = jnp.maximum(xx2 - xx1, jnp.float32(0.0)) * jnp.maximum(
                    yy2 - yy1, jnp.float32(0.0))
                a2 = (xb - xa) * (yb - ya)
                iou = inter / (a1 + a2 - inter + jnp.float32(1e-9))
                gli = iota_i + (base + 16 * j)
                supp = (iou > jnp.float32(0.3)) | (gli == iiv)
                msc[sl] = jnp.where(supp, NEG, m)

        def it(tp, carry):
            one_step(tp * 2, redv0, redi0)
            one_step(tp * 2 + 1, redv1, redi1)
            return carry

        lax.fori_loop(0, NKEEP // 2, it, 0)

        gbase = pl.multiple_of(wid * 8, 8)
        idxs = kidx.at[pl.ds(gbase, 8)]
        pltpu.async_copy(roi_h.at[idxs], rows, sem).wait()
        pltpu.sync_copy(rows, feats_o.at[pl.ds(gbase, 8)])

        @pl.when(wid == 0)
        def _():
            pltpu.sync_copy(kval, kval_o)
            pltpu.sync_copy(kbox, kbox_o)

    return k(scores_p, x1p, y1p, x2p, y2p, roi)


def _tc_body(feats, mean2, comps, mT, kval, kbox, ml, kbw,
             boxes_o, scores_o, pred_o):
    fm = feats[...] - mean2[...]
    f = jnp.dot(fm, comps[...], preferred_element_type=jnp.float32) / ml[0, 0]
    h2 = kbw[0, 0] * kbw[0, 0]
    fsq = 0.5 * jnp.sum(f * f, axis=1, keepdims=True)

    def step(c, carry):
        M, S = carry
        off = pl.multiple_of(c * KCH, KCH)
        mc = mT[:, pl.ds(off, KCH)]
        dotp = jnp.dot(f, mc, preferred_element_type=jnp.float32)
        msq = 0.5 * jnp.sum(mc * mc, axis=0, keepdims=True)
        tt = (dotp - msq) / h2
        col = lax.broadcasted_iota(jnp.int32, (KPAD, KCH), 1) + c * KCH
        tt = jnp.where(col < NTRAIN, tt, NEG)
        cm = jnp.max(tt, axis=1, keepdims=True)
        newM = jnp.maximum(M, cm)
        S = S * jnp.exp(M - newM) + jnp.sum(jnp.exp(tt - newM), axis=1, keepdims=True)
        return (newM, S)

    M0 = jnp.full((KPAD, 1), NEG, jnp.float32)
    S0 = jnp.zeros((KPAD, 1), jnp.float32)
    M, S = lax.fori_loop(0, NTPAD // KCH, step, (M0, S0))
    log_dens = (M + jnp.log(S) - fsq / h2
                - jnp.float32(np.log(NTRAIN))
                - jnp.float32(0.5 * NPCA) * jnp.log(jnp.float32(2.0 * np.pi) * h2))
    prob = 1.0 / (1.0 + jnp.exp(jnp.float32(0.05) * (log_dens - jnp.float32(12.0))))
    vm = kval[...] > jnp.float32(0.5)
    sc = jnp.where(vm, prob, jnp.float32(0.0))
    scores_o[...] = sc
    boxes_o[...] = jnp.where(vm, kbox[...], jnp.float32(0.0))
    pred_o[...] = jnp.max(sc).reshape(1, 1)


def _tc_dense(feats, mean2, comps, mT, kval2, kbox2, ml2, kb2):
    return pl.pallas_call(
        _tc_body,
        out_shape=[
            jax.ShapeDtypeStruct((KPAD, 4), jnp.float32),
            jax.ShapeDtypeStruct((KPAD, 1), jnp.float32),
            jax.ShapeDtypeStruct((1, 1), jnp.float32),
        ],
    )(feats, mean2, comps, mT, kval2, kbox2, ml2, kb2)


def kernel(boxes, scores, roi_features, pca_mean, pca_components,
           max_length, kde_memory, kde_bandwidth):
    pad = NPAD - NBOX
    scores_p = jnp.concatenate([scores, jnp.full((pad,), NEG)])
    bp = jnp.pad(boxes, ((0, pad), (0, 0)))
    x1p, y1p, x2p, y2p = bp[:, 0], bp[:, 1], bp[:, 2], bp[:, 3]

    feats, kval, kboxf = _sc_nms_gather(scores_p, x1p, y1p, x2p, y2p, roi_features)

    mT = jnp.pad(kde_memory.T, ((0, 0), (0, NTPAD - NTRAIN)))
    boxes_o, scores_o, pred_o = _tc_dense(
        feats,
        pca_mean.reshape(1, FEAT),
        pca_components,
        mT,
        kval.reshape(KPAD, 1),
        kboxf.reshape(KPAD, 4),
        max_length.reshape(1, 1),
        kde_bandwidth.reshape(1, 1),
    )
    return boxes_o[:NKEEP], scores_o[:NKEEP, 0], pred_o[0, 0]


# speedup vs baseline: 18.6216x; 1.1778x over previous
import functools

import jax
import jax.numpy as jnp
import numpy as np
from jax import lax
from jax.experimental import pallas as pl
from jax.experimental.pallas import tpu as pltpu
from jax.experimental.pallas import tpu_sc as plsc

NBOX = 5000
NPAD = 5120
CHUNK = 320
NVR = CHUNK // 16
NSUB = 16
NKEEP = 100
KPAD = 128
FEAT = 4096
NTRAIN = 40000
NTPAD = 40960
KCH = 4096
NPCA = 16
NEG = -1e30


def _sc_nms_gather(scores_p, x1p, y1p, x2p, y2p, roi):
    mesh = plsc.VectorSubcoreMesh(
        core_axis_name="c", subcore_axis_name="s", num_cores=1, num_subcores=NSUB
    )

    @functools.partial(
        pl.kernel,
        out_type=[
            jax.ShapeDtypeStruct((KPAD, FEAT), jnp.float32),
            jax.ShapeDtypeStruct((KPAD,), jnp.float32),
            jax.ShapeDtypeStruct((KPAD * 4,), jnp.float32),
        ],
        mesh=mesh,
        compiler_params=pltpu.CompilerParams(needs_layout_passes=False),
        scratch_types=[
            pltpu.VMEM((NPAD,), jnp.float32),
            pltpu.VMEM((NPAD,), jnp.float32),
            pltpu.VMEM((NPAD,), jnp.float32),
            pltpu.VMEM((NPAD,), jnp.float32),
            pltpu.VMEM((CHUNK,), jnp.float32),
            pltpu.VMEM((CHUNK,), jnp.float32),
            pltpu.VMEM((CHUNK,), jnp.float32),
            pltpu.VMEM((CHUNK,), jnp.float32),
            pltpu.VMEM((CHUNK,), jnp.float32),
            pltpu.VMEM((KPAD,), jnp.int32),
            pltpu.VMEM((KPAD,), jnp.float32),
            pltpu.VMEM((KPAD * 4,), jnp.float32),
            pltpu.VMEM_SHARED((NSUB * 16,), jnp.float32),
            pltpu.VMEM_SHARED((NSUB * 16,), jnp.float32),
            pltpu.VMEM((16,), jnp.float32),
            pltpu.VMEM((NSUB * 16,), jnp.float32),
            pltpu.VMEM((8, FEAT), jnp.float32),
            pltpu.SemaphoreType.DMA,
        ],
    )
    def k(scores_h, x1_h, y1_h, x2_h, y2_h, roi_h,
          feats_o, kval_o, kbox_o,
          x1f, y1f, x2f, y2f, x1c, y1c, x2c, y2c, msc,
          kidx, kval, kbox, redv0, redv1, wv, rv2, rows, sem):
        wid = lax.axis_index("s")
        base = pl.multiple_of(wid * CHUNK, CHUNK)

        pltpu.sync_copy(x1_h, x1f)
        pltpu.sync_copy(y1_h, y1f)
        pltpu.sync_copy(x2_h, x2f)
        pltpu.sync_copy(y2_h, y2f)
        pltpu.sync_copy(scores_h.at[pl.ds(base, CHUNK)], msc)
        pltpu.sync_copy(x1_h.at[pl.ds(base, CHUNK)], x1c)
        pltpu.sync_copy(y1_h.at[pl.ds(base, CHUNK)], y1c)
        pltpu.sync_copy(x2_h.at[pl.ds(base, CHUNK)], x2c)
        pltpu.sync_copy(y2_h.at[pl.ds(base, CHUNK)], y2c)

        iota_i = lax.iota(jnp.int32, 16)
        iota_f = iota_i.astype(jnp.float32)
        zeros_i = jnp.zeros((16,), jnp.int32)
        base_f = lax.convert_element_type(base, jnp.float32)

        for j in range(NVR):
            sl = pl.ds(16 * j, 16)
            s = msc[sl]
            w = x2c[sl] - x1c[sl]
            h = y2c[sl] - y1c[sl]
            ok = (s >= jnp.float32(0.001)) & (w >= jnp.float32(100.0)) & (h >= jnp.float32(100.0))
            msc[sl] = jnp.where(ok, s, NEG)

        zf = jnp.zeros((16,), jnp.float32)
        zi = jnp.zeros((16,), jnp.int32)
        for kk in range(KPAD // 16):
            kval[pl.ds(16 * kk, 16)] = zf
            kidx[pl.ds(16 * kk, 16)] = zi
        for kk in range(KPAD * 4 // 16):
            kbox[pl.ds(16 * kk, 16)] = zf

        def one_step(t, redv):
            bv = jnp.full((16,), jnp.float32(-3e38))
            bj = zf
            for j in range(NVR):
                v = msc[pl.ds(16 * j, 16)]
                take = v > bv
                bv = jnp.where(take, v, bv)
                bj = jnp.where(take, jnp.full((16,), jnp.float32(j)), bj)
            lm = jnp.max(bv)
            gidx = base_f + bj * jnp.float32(16.0) + iota_f
            li = jnp.min(jnp.where(bv == lm, gidx, jnp.float32(3e38)))

            wv[...] = jnp.where(iota_i < 8, jnp.full((16,), lm), jnp.full((16,), li))
            woff = pl.multiple_of(wid * 16, 16)
            pltpu.sync_copy(wv, redv.at[pl.ds(woff, 16)])
            plsc.subcore_barrier()
            pltpu.sync_copy(redv, rv2)
            gv = plsc.load_gather(rv2, [iota_i * 16])
            gi = plsc.load_gather(rv2, [iota_i * 16 + 8])
            G = jnp.max(gv)
            If = jnp.min(jnp.where(gv == G, gi, jnp.float32(3e38)))
            ii = If.astype(jnp.int32)
            ii = lax.max(0, lax.min(ii, NPAD - 1))
            iiv = jnp.full((16,), ii)

            bx1 = plsc.load_gather(x1f, [iiv])
            by1 = plsc.load_gather(y1f, [iiv])
            bx2 = plsc.load_gather(x2f, [iiv])
            by2 = plsc.load_gather(y2f, [iiv])

            validf = jnp.where(G > jnp.float32(-5e29), jnp.float32(1.0), jnp.float32(0.0))
            tv = jnp.full((16,), t)
            plsc.store_scatter(kval, [tv], jnp.full((16,), validf), mask=iota_i == 0)
            plsc.store_scatter(kidx, [tv], iiv, mask=iota_i == 0)
            cvec = jnp.where(iota_i == 0, bx1,
                             jnp.where(iota_i == 1, by1,
                                       jnp.where(iota_i == 2, bx2, by2)))
            plsc.store_scatter(kbox, [tv * 4 + iota_i], cvec, mask=iota_i < 4)

            a1 = (bx2 - bx1) * (by2 - by1)
            for j in range(NVR):
                sl = pl.ds(16 * j, 16)
                xa = x1c[sl]
                ya = y1c[sl]
                xb = x2c[sl]
                yb = y2c[sl]
                m = msc[sl]
                xx1 = jnp.maximum(bx1, xa)
                yy1 = jnp.maximum(by1, ya)
                xx2 = jnp.minimum(bx2, xb)
                yy2 = jnp.minimum(by2, yb)
                inter = jnp.maximum(xx2 - xx1, jnp.float32(0.0)) * jnp.maximum(
                    yy2 - yy1, jnp.float32(0.0))
                a2 = (xb - xa) * (yb - ya)
                iou = inter / (a1 + a2 - inter + jnp.float32(1e-9))
                gli = iota_i + (base + 16 * j)
                supp = (iou > jnp.float32(0.3)) | (gli == iiv)
                msc[sl] = jnp.where(supp, NEG, m)

        def it(tp, carry):
            one_step(tp * 2, redv0)
            one_step(tp * 2 + 1, redv1)
            return carry

        lax.fori_loop(0, NKEEP // 2, it, 0)

        gbase = pl.multiple_of(wid * 8, 8)
        idxs = kidx.at[pl.ds(gbase, 8)]
        pltpu.async_copy(roi_h.at[idxs], rows, sem).wait()
        pltpu.sync_copy(rows, feats_o.at[pl.ds(gbase, 8)])

        @pl.when(wid == 0)
        def _():
            pltpu.sync_copy(kval, kval_o)
            pltpu.sync_copy(kbox, kbox_o)

    return k(scores_p, x1p, y1p, x2p, y2p, roi)


def _tc_body(feats, mean2, comps, mT, kval, kbox, ml, kbw,
             boxes_o, scores_o, pred_o):
    fm = feats[...] - mean2[...]
    f = jnp.dot(fm, comps[...], preferred_element_type=jnp.float32) / ml[0, 0]
    h2 = kbw[0, 0] * kbw[0, 0]
    fsq = 0.5 * jnp.sum(f * f, axis=1, keepdims=True)

    def step(c, carry):
        M, S = carry
        off = pl.multiple_of(c * KCH, KCH)
        mc = mT[:, pl.ds(off, KCH)]
        dotp = jnp.dot(f, mc, preferred_element_type=jnp.float32)
        msq = 0.5 * jnp.sum(mc * mc, axis=0, keepdims=True)
        tt = (dotp - msq) / h2
        col = lax.broadcasted_iota(jnp.int32, (KPAD, KCH), 1) + c * KCH
        tt = jnp.where(col < NTRAIN, tt, NEG)
        cm = jnp.max(tt, axis=1, keepdims=True)
        newM = jnp.maximum(M, cm)
        S = S * jnp.exp(M - newM) + jnp.sum(jnp.exp(tt - newM), axis=1, keepdims=True)
        return (newM, S)

    M0 = jnp.full((KPAD, 1), NEG, jnp.float32)
    S0 = jnp.zeros((KPAD, 1), jnp.float32)
    M, S = lax.fori_loop(0, NTPAD // KCH, step, (M0, S0))
    log_dens = (M + jnp.log(S) - fsq / h2
                - jnp.float32(np.log(NTRAIN))
                - jnp.float32(0.5 * NPCA) * jnp.log(jnp.float32(2.0 * np.pi) * h2))
    prob = 1.0 / (1.0 + jnp.exp(jnp.float32(0.05) * (log_dens - jnp.float32(12.0))))
    vm = kval[...] > jnp.float32(0.5)
    sc = jnp.where(vm, prob, jnp.float32(0.0))
    scores_o[...] = sc
    boxes_o[...] = jnp.where(vm, kbox[...], jnp.float32(0.0))
    pred_o[...] = jnp.max(sc).reshape(1, 1)


def _tc_dense(feats, mean2, comps, mT, kval2, kbox2, ml2, kb2):
    return pl.pallas_call(
        _tc_body,
        out_shape=[
            jax.ShapeDtypeStruct((KPAD, 4), jnp.float32),
            jax.ShapeDtypeStruct((KPAD, 1), jnp.float32),
            jax.ShapeDtypeStruct((1, 1), jnp.float32),
        ],
    )(feats, mean2, comps, mT, kval2, kbox2, ml2, kb2)


def kernel(boxes, scores, roi_features, pca_mean, pca_components,
           max_length, kde_memory, kde_bandwidth):
    pad = NPAD - NBOX
    scores_p = jnp.concatenate([scores, jnp.full((pad,), NEG)])
    bp = jnp.pad(boxes, ((0, pad), (0, 0)))
    x1p, y1p, x2p, y2p = bp[:, 0], bp[:, 1], bp[:, 2], bp[:, 3]

    feats, kval, kboxf = _sc_nms_gather(scores_p, x1p, y1p, x2p, y2p, roi_features)

    mT = jnp.pad(kde_memory.T, ((0, 0), (0, NTPAD - NTRAIN)))
    boxes_o, scores_o, pred_o = _tc_dense(
        feats,
        pca_mean.reshape(1, FEAT),
        pca_components,
        mT,
        kval.reshape(KPAD, 1),
        kboxf.reshape(KPAD, 4),
        max_length.reshape(1, 1),
        kde_bandwidth.reshape(1, 1),
    )
    return boxes_o[:NKEEP], scores_o[:NKEEP, 0], pred_o[0, 0]
